# baseline (speedup 1.0000x reference)
import functools

import jax
import jax.numpy as jnp
from jax import lax
from jax.experimental import pallas as pl
from jax.experimental.pallas import tpu as pltpu
from jax.experimental.pallas import tpu_sc as plsc

BATCH = 16384
EMBED_DIM = 32
HIDDEN = 64

_NC = 2
_NS = 16
_NW = _NC * _NS
_B_PER_W = BATCH // _NW
_CHUNK = 128
_NCHUNK = _B_PER_W // _CHUNK


def _sc_gather_body(uid_hbm, pid_hbm, iid_hbm, ut_hbm, pt_hbm, it_hbm,
                    u_out, p_out, i_out,
                    idx_u, idx_p, idx_i, rows_u, rows_p, rows_i, sem):
    wid = lax.axis_index("s") * _NC + lax.axis_index("c")
    row0 = wid * _NCHUNK
    base = wid * _B_PER_W

    pltpu.sync_copy(uid_hbm.at[pl.ds(row0, _NCHUNK)], idx_u)
    pltpu.sync_copy(pid_hbm.at[pl.ds(row0, _NCHUNK)], idx_p)
    pltpu.sync_copy(iid_hbm.at[pl.ds(row0, _NCHUNK)], idx_i)

    copies = []
    for c in range(_NCHUNK):
        sl = pl.ds(c * _CHUNK, _CHUNK)
        copies.append(pltpu.async_copy(ut_hbm.at[idx_u.at[c]], rows_u.at[sl], sem))
        copies.append(pltpu.async_copy(pt_hbm.at[idx_p.at[c]], rows_p.at[sl], sem))
        copies.append(pltpu.async_copy(it_hbm.at[idx_i.at[c]], rows_i.at[sl], sem))
    for cp in copies:
        cp.wait()

    out_sl = pl.ds(base, _B_PER_W)
    pltpu.sync_copy(rows_u, u_out.at[out_sl])
    pltpu.sync_copy(rows_p, p_out.at[out_sl])
    pltpu.sync_copy(rows_i, i_out.at[out_sl])


@jax.jit
def _sc_gather(uids2, pids2, iids2, user_table, product_table, interaction_table):
    mesh = plsc.VectorSubcoreMesh(core_axis_name="c", subcore_axis_name="s")
    f = pl.kernel(
        _sc_gather_body,
        out_type=[jax.ShapeDtypeStruct((BATCH, EMBED_DIM), jnp.float32)] * 3,
        mesh=mesh,
        scratch_types=[
            pltpu.VMEM((_NCHUNK, _CHUNK), jnp.int32),
            pltpu.VMEM((_NCHUNK, _CHUNK), jnp.int32),
            pltpu.VMEM((_NCHUNK, _CHUNK), jnp.int32),
            pltpu.VMEM((_B_PER_W, EMBED_DIM), jnp.float32),
            pltpu.VMEM((_B_PER_W, EMBED_DIM), jnp.float32),
            pltpu.VMEM((_B_PER_W, EMBED_DIM), jnp.float32),
            pltpu.SemaphoreType.DMA,
        ],
        compiler_params=pltpu.CompilerParams(use_tc_tiling_on_sc=False),
    )
    return f(uids2, pids2, iids2, user_table, product_table, interaction_table)


def _mlp_body(u_ref, p_ref, i_ref, w1_ref, b1_ref, w2_ref, b2_ref, o_ref):
    h = jnp.dot(u_ref[...], w1_ref[0:EMBED_DIM, :],
                preferred_element_type=jnp.float32)
    h = h + jnp.dot(p_ref[...], w1_ref[EMBED_DIM:2 * EMBED_DIM, :],
                    preferred_element_type=jnp.float32)
    h = h + jnp.dot(i_ref[...], w1_ref[2 * EMBED_DIM:3 * EMBED_DIM, :],
                    preferred_element_type=jnp.float32)
    h = jnp.maximum(h + b1_ref[...], 0.0)
    o_ref[...] = jnp.dot(h, w2_ref[...],
                         preferred_element_type=jnp.float32) + b2_ref[...]


_MLP_BLK = 4096


@jax.jit
def _mlp(u, p, i, W1, b1, W2, b2):
    grid = (BATCH // _MLP_BLK,)
    return pl.pallas_call(
        _mlp_body,
        grid=grid,
        in_specs=[
            pl.BlockSpec((_MLP_BLK, EMBED_DIM), lambda g: (g, 0)),
            pl.BlockSpec((_MLP_BLK, EMBED_DIM), lambda g: (g, 0)),
            pl.BlockSpec((_MLP_BLK, EMBED_DIM), lambda g: (g, 0)),
            pl.BlockSpec((3 * EMBED_DIM, HIDDEN), lambda g: (0, 0)),
            pl.BlockSpec((1, HIDDEN), lambda g: (0, 0)),
            pl.BlockSpec((HIDDEN, 1), lambda g: (0, 0)),
            pl.BlockSpec((1, 1), lambda g: (0, 0)),
        ],
        out_specs=pl.BlockSpec((_MLP_BLK, 1), lambda g: (g, 0)),
        out_shape=jax.ShapeDtypeStruct((BATCH, 1), jnp.float32),
    )(u, p, i, W1, b1, W2, b2)


def kernel(user_ids, product_ids, interaction_ids, user_table, product_table,
           interaction_table, W1, b1, W2, b2):
    uids2 = user_ids.astype(jnp.int32).reshape(BATCH // _CHUNK, _CHUNK)
    pids2 = product_ids.astype(jnp.int32).reshape(BATCH // _CHUNK, _CHUNK)
    iids2 = interaction_ids.astype(jnp.int32).reshape(BATCH // _CHUNK, _CHUNK)
    u, p, i = _sc_gather(uids2, pids2, iids2, user_table, product_table,
                         interaction_table)
    return _mlp(u, p, i, W1, b1.reshape(1, HIDDEN), W2, b2.reshape(1, 1))


# speedup vs baseline: 1.8495x; 1.8495x over previous
import jax
import jax.numpy as jnp
from jax import lax
from jax.experimental import pallas as pl
from jax.experimental.pallas import tpu as pltpu
from jax.experimental.pallas import tpu_sc as plsc

BATCH = 16384
EMBED_DIM = 32
HIDDEN = 64
N_INTER = 1000

_NC = 2
_NS = 16
_NW = _NC * _NS
_B_PER_W = BATCH // _NW
_GRP = 16
_NGRP = _B_PER_W // _GRP
_LANES = 128


def _sc_gather_body(uid_hbm, pid_hbm, iid_hbm, ut_hbm, pt_hbm, it_hbm,
                    u_out, p_out, i_out,
                    ids_v, slab_v, itab_v, rows_v, sem):
    wid = lax.axis_index("s") * _NC + lax.axis_index("c")
    base = wid * _B_PER_W
    rbase = base * EMBED_DIM

    e_lo = lax.iota(jnp.int32, 16)
    e_hi = e_lo + 16

    def gather_big(ids_hbm, tab_hbm, out_hbm):
        pltpu.sync_copy(ids_hbm.at[pl.ds(base, _B_PER_W)], ids_v)

        def chunk(g, _):
            vv = ids_v[pl.ds(g * _GRP, _GRP)]
            for j in range(_GRP):
                tile_col = (vv[j] // _LANES) * _LANES
                pltpu.make_async_copy(
                    tab_hbm.at[:, pl.ds(tile_col, _LANES)],
                    slab_v.at[:, pl.ds(j * _LANES, _LANES)],
                    sem,
                ).start()
            pltpu.make_async_copy(
                tab_hbm.at[:, pl.ds(0, _GRP * _LANES)], slab_v, sem
            ).wait()
            for j in range(_GRP):
                lane = lax.rem(vv[j], _LANES) + j * _LANES
                l_idx = jnp.zeros((16,), jnp.int32) + lane
                lo = plsc.load_gather(slab_v, [e_lo, l_idx])
                hi = plsc.load_gather(slab_v, [e_hi, l_idx])
                r0 = (g * _GRP + j) * EMBED_DIM
                rows_v[pl.ds(r0, 16)] = lo
                rows_v[pl.ds(r0 + 16, 16)] = hi
            return _

        lax.fori_loop(0, _NGRP, chunk, 0)
        pltpu.sync_copy(rows_v, out_hbm.at[pl.ds(rbase, _B_PER_W * EMBED_DIM)])

    gather_big(uid_hbm, ut_hbm, u_out)
    gather_big(pid_hbm, pt_hbm, p_out)

    pltpu.sync_copy(it_hbm, itab_v)
    pltpu.sync_copy(iid_hbm.at[pl.ds(base, _B_PER_W)], ids_v)

    def ichunk(g, _):
        vv = ids_v[pl.ds(g * _GRP, _GRP)]
        for j in range(_GRP):
            l_idx = jnp.zeros((16,), jnp.int32) + vv[j]
            lo = plsc.load_gather(itab_v, [e_lo, l_idx])
            hi = plsc.load_gather(itab_v, [e_hi, l_idx])
            r0 = (g * _GRP + j) * EMBED_DIM
            rows_v[pl.ds(r0, 16)] = lo
            rows_v[pl.ds(r0 + 16, 16)] = hi
        return _

    lax.fori_loop(0, _NGRP, ichunk, 0)
    pltpu.sync_copy(rows_v, i_out.at[pl.ds(rbase, _B_PER_W * EMBED_DIM)])


@jax.jit
def _sc_gather(user_ids, product_ids, interaction_ids, ut_t, pt_t, it_t):
    mesh = plsc.VectorSubcoreMesh(core_axis_name="c", subcore_axis_name="s")
    f = pl.kernel(
        _sc_gather_body,
        out_type=[jax.ShapeDtypeStruct((BATCH * EMBED_DIM,), jnp.float32)] * 3,
        mesh=mesh,
        scratch_types=[
            pltpu.VMEM((_B_PER_W,), jnp.int32),
            pltpu.VMEM((EMBED_DIM, _GRP * _LANES), jnp.float32),
            pltpu.VMEM((EMBED_DIM, N_INTER), jnp.float32),
            pltpu.VMEM((_B_PER_W * EMBED_DIM,), jnp.float32),
            pltpu.SemaphoreType.DMA,
        ],
        compiler_params=pltpu.CompilerParams(
            use_tc_tiling_on_sc=True, needs_layout_passes=False),
    )
    return f(user_ids, product_ids, interaction_ids, ut_t, pt_t, it_t)


def _mlp_body(u_ref, p_ref, i_ref, w1_ref, b1_ref, w2_ref, b2_ref, o_ref):
    h = jnp.dot(u_ref[...], w1_ref[0:EMBED_DIM, :],
                preferred_element_type=jnp.float32)
    h = h + jnp.dot(p_ref[...], w1_ref[EMBED_DIM:2 * EMBED_DIM, :],
                    preferred_element_type=jnp.float32)
    h = h + jnp.dot(i_ref[...], w1_ref[2 * EMBED_DIM:3 * EMBED_DIM, :],
                    preferred_element_type=jnp.float32)
    h = jnp.maximum(h + b1_ref[...], 0.0)
    o_ref[...] = jnp.dot(h, w2_ref[...],
                         preferred_element_type=jnp.float32) + b2_ref[...]


_MLP_BLK = 4096


@jax.jit
def _mlp(u, p, i, W1, b1, W2, b2):
    grid = (BATCH // _MLP_BLK,)
    return pl.pallas_call(
        _mlp_body,
        grid=grid,
        in_specs=[
            pl.BlockSpec((_MLP_BLK, EMBED_DIM), lambda g: (g, 0)),
            pl.BlockSpec((_MLP_BLK, EMBED_DIM), lambda g: (g, 0)),
            pl.BlockSpec((_MLP_BLK, EMBED_DIM), lambda g: (g, 0)),
            pl.BlockSpec((3 * EMBED_DIM, HIDDEN), lambda g: (0, 0)),
            pl.BlockSpec((1, HIDDEN), lambda g: (0, 0)),
            pl.BlockSpec((HIDDEN, 1), lambda g: (0, 0)),
            pl.BlockSpec((1, 1), lambda g: (0, 0)),
        ],
        out_specs=pl.BlockSpec((_MLP_BLK, 1), lambda g: (g, 0)),
        out_shape=jax.ShapeDtypeStruct((BATCH, 1), jnp.float32),
    )(u, p, i, W1, b1, W2, b2)


def kernel(user_ids, product_ids, interaction_ids, user_table, product_table,
           interaction_table, W1, b1, W2, b2):
    uids = user_ids.astype(jnp.int32)
    pids = product_ids.astype(jnp.int32)
    iids = interaction_ids.astype(jnp.int32)
    u_f, p_f, i_f = _sc_gather(uids, pids, iids, user_table.T,
                               product_table.T, interaction_table.T)
    u = u_f.reshape(BATCH, EMBED_DIM)
    p = p_f.reshape(BATCH, EMBED_DIM)
    i = i_f.reshape(BATCH, EMBED_DIM)
    return _mlp(u, p, i, W1, b1.reshape(1, HIDDEN), W2, b2.reshape(1, 1))


# speedup vs baseline: 2.0263x; 1.0956x over previous
import jax
import jax.numpy as jnp
from jax import lax
from jax.experimental import pallas as pl
from jax.experimental.pallas import tpu as pltpu
from jax.experimental.pallas import tpu_sc as plsc

BATCH = 16384
EMBED_DIM = 32
HIDDEN = 64
N_INTER = 1000

_NC = 2
_NS = 16
_NW = _NC * _NS
_B_PER_W = BATCH // _NW
_CH = 8
_NCH = _B_PER_W // _CH
_LANES = 128
_ROWS = 128


def _sc_gather_body(uid_hbm, pid_hbm, iid_hbm, ut_hbm, pt_hbm, it_hbm,
                    u_out, p_out, i_out,
                    ids_v, slab0, slab1, itab_v, rows_v, sem0, sem1):
    wid = lax.axis_index("s") * _NC + lax.axis_index("c")
    base = wid * _B_PER_W

    e_lo = lax.iota(jnp.int32, 16)
    e_hi = e_lo + 16

    def gather_big(ids_hbm, tab_hbm, out_hbm):
        pltpu.sync_copy(ids_hbm.at[pl.ds(base, _B_PER_W)], ids_v)

        def fire(vv, j0, slab, sem):
            for j in range(_CH):
                tile_col = (vv[j0 + j] // _LANES) * _LANES
                pltpu.make_async_copy(
                    tab_hbm.at[:, pl.ds(tile_col, _LANES)],
                    slab.at[:, pl.ds(j * _LANES, _LANES)],
                    sem,
                ).start()

        def drain(slab, sem):
            pltpu.make_async_copy(
                tab_hbm.at[:, pl.ds(0, _CH * _LANES)], slab, sem
            ).wait()

        def extract(vv, j0, slab, cbase):
            for j in range(_CH):
                lane = lax.rem(vv[j0 + j], _LANES) + j * _LANES
                l_idx = jnp.zeros((16,), jnp.int32) + lane
                lo = plsc.load_gather(slab, [e_lo, l_idx])
                hi = plsc.load_gather(slab, [e_hi, l_idx])
                r_idx = jnp.zeros((16,), jnp.int32) + lax.rem(cbase + j, _ROWS)
                plsc.store_scatter(rows_v, [r_idx, e_lo], lo)
                plsc.store_scatter(rows_v, [r_idx, e_hi], hi)

        vv0 = ids_v[pl.ds(0, 16)]
        fire(vv0, 0, slab0, sem0)

        def body(g, _):
            vv = ids_v[pl.ds(g * 16, 16)]
            fire(vv, _CH, slab1, sem1)
            drain(slab0, sem0)
            extract(vv, 0, slab0, g * 16)

            @pl.when(g < _NCH // 2 - 1)
            def _fire_next():
                vvn = ids_v[pl.ds(g * 16 + 16, 16)]
                fire(vvn, 0, slab0, sem0)

            drain(slab1, sem1)
            extract(vv, _CH, slab1, g * 16 + _CH)

            @pl.when(lax.rem(g, 8) == 7)
            def _flush():
                pltpu.sync_copy(
                    rows_v,
                    out_hbm.at[pl.ds(base + (g // 8) * _ROWS, _ROWS), :])

            return _

        lax.fori_loop(0, _NCH // 2, body, 0)

    gather_big(uid_hbm, ut_hbm, u_out)
    gather_big(pid_hbm, pt_hbm, p_out)

    pltpu.sync_copy(it_hbm, itab_v)
    pltpu.sync_copy(iid_hbm.at[pl.ds(base, _B_PER_W)], ids_v)

    def ichunk(g, _):
        vv = ids_v[pl.ds(g * 16, 16)]
        for j in range(16):
            l_idx = jnp.zeros((16,), jnp.int32) + vv[j]
            lo = plsc.load_gather(itab_v, [e_lo, l_idx])
            hi = plsc.load_gather(itab_v, [e_hi, l_idx])
            r_idx = jnp.zeros((16,), jnp.int32) + lax.rem(g * 16 + j, _ROWS)
            plsc.store_scatter(rows_v, [r_idx, e_lo], lo)
            plsc.store_scatter(rows_v, [r_idx, e_hi], hi)

        @pl.when(lax.rem(g, 8) == 7)
        def _flush():
            pltpu.sync_copy(
                rows_v, i_out.at[pl.ds(base + (g // 8) * _ROWS, _ROWS), :])

        return _

    lax.fori_loop(0, _B_PER_W // 16, ichunk, 0)


@jax.jit
def _sc_gather(user_ids, product_ids, interaction_ids, ut_t, pt_t, it_t):
    mesh = plsc.VectorSubcoreMesh(core_axis_name="c", subcore_axis_name="s")
    f = pl.kernel(
        _sc_gather_body,
        out_type=[jax.ShapeDtypeStruct((BATCH, EMBED_DIM), jnp.float32)] * 3,
        mesh=mesh,
        scratch_types=[
            pltpu.VMEM((_B_PER_W,), jnp.int32),
            pltpu.VMEM((EMBED_DIM, _CH * _LANES), jnp.float32),
            pltpu.VMEM((EMBED_DIM, _CH * _LANES), jnp.float32),
            pltpu.VMEM((EMBED_DIM, N_INTER), jnp.float32),
            pltpu.VMEM((_ROWS, EMBED_DIM), jnp.float32),
            pltpu.SemaphoreType.DMA,
            pltpu.SemaphoreType.DMA,
        ],
        compiler_params=pltpu.CompilerParams(
            use_tc_tiling_on_sc=True, needs_layout_passes=False),
    )
    return f(user_ids, product_ids, interaction_ids, ut_t, pt_t, it_t)


def _mlp_body(u_ref, p_ref, i_ref, w1_ref, b1_ref, w2_ref, b2_ref, o_ref):
    h = jnp.dot(u_ref[...], w1_ref[0:EMBED_DIM, :],
                preferred_element_type=jnp.float32)
    h = h + jnp.dot(p_ref[...], w1_ref[EMBED_DIM:2 * EMBED_DIM, :],
                    preferred_element_type=jnp.float32)
    h = h + jnp.dot(i_ref[...], w1_ref[2 * EMBED_DIM:3 * EMBED_DIM, :],
                    preferred_element_type=jnp.float32)
    h = jnp.maximum(h + b1_ref[...], 0.0)
    o_ref[...] = jnp.dot(h, w2_ref[...],
                         preferred_element_type=jnp.float32) + b2_ref[...]


_MLP_BLK = 4096


@jax.jit
def _mlp(u, p, i, W1, b1, W2, b2):
    grid = (BATCH // _MLP_BLK,)
    return pl.pallas_call(
        _mlp_body,
        grid=grid,
        in_specs=[
            pl.BlockSpec((_MLP_BLK, EMBED_DIM), lambda g: (g, 0)),
            pl.BlockSpec((_MLP_BLK, EMBED_DIM), lambda g: (g, 0)),
            pl.BlockSpec((_MLP_BLK, EMBED_DIM), lambda g: (g, 0)),
            pl.BlockSpec((3 * EMBED_DIM, HIDDEN), lambda g: (0, 0)),
            pl.BlockSpec((1, HIDDEN), lambda g: (0, 0)),
            pl.BlockSpec((HIDDEN, 1), lambda g: (0, 0)),
            pl.BlockSpec((1, 1), lambda g: (0, 0)),
        ],
        out_specs=pl.BlockSpec((_MLP_BLK, 1), lambda g: (g, 0)),
        out_shape=jax.ShapeDtypeStruct((BATCH, 1), jnp.float32),
    )(u, p, i, W1, b1, W2, b2)


def kernel(user_ids, product_ids, interaction_ids, user_table, product_table,
           interaction_table, W1, b1, W2, b2):
    uids = user_ids.astype(jnp.int32)
    pids = product_ids.astype(jnp.int32)
    iids = interaction_ids.astype(jnp.int32)
    u, p, i = _sc_gather(uids, pids, iids, user_table.T,
                         product_table.T, interaction_table.T)
    return _mlp(u, p, i, W1, b1.reshape(1, HIDDEN), W2, b2.reshape(1, 1))
